# Initial kernel scaffold; baseline (speedup 1.0000x reference)
import jax
import jax.numpy as jnp
from jax.experimental import pallas as pl


def kernel(x_users, x_items, edge_index, W_users, b_users, W_items, b_items):
    raise NotImplementedError("write your pallas kernel here")


# speedup vs baseline: 64.5880x; 64.5880x over previous
import functools

import jax
import jax.numpy as jnp
import numpy as np
from jax import lax
from jax.experimental import pallas as pl
from jax.experimental.pallas import tpu as pltpu
from jax.experimental.pallas import tpu_sc as plsc

N_USERS = 50000
N_ITEMS = 50000
E = 800000
C = 4
FIN = 16
D = 16
F = C * D
HF = F // 2

NC = 2
NS = 16
L = 16

CHUNK = 128
E_PAD = ((E + NS * CHUNK - 1) // (NS * CHUNK)) * (NS * CHUNK)
EPT = E_PAD // NS
NCHUNK = EPT // CHUNK

ACC_ROWS = 50048
CNT_H = N_ITEMS // 2
CNT_ROWS = 25088
CL = 32
ACC_CH = ACC_ROWS // CHUNK
CNT_CH = CNT_ROWS // CHUNK


def _positional_encoding_flat():
    pos = np.arange(C, dtype=np.float32)[:, None]
    i = np.arange(D, dtype=np.float32)[None, :]
    angle = pos / np.power(10000.0, (2.0 * np.floor(i / 2.0)) / D)
    pe = np.zeros((C, D), dtype=np.float32)
    pe[:, 0::2] = np.sin(angle[:, 0::2])
    pe[:, 1::2] = np.cos(angle[:, 1::2])
    return pe.reshape(F)


def _agg_body(xa, xb, src_hbm, dst_hbm, agg_out,
              srcv, dst2, rows, acc_sh, sem):
    cid = lax.axis_index("c")
    sid = lax.axis_index("s")

    zv = jnp.zeros((L,), jnp.float32)

    def fill(i, _):
        rows[i, pl.ds(0, L)] = zv
        rows[i, pl.ds(L, L)] = zv
        return 0

    lax.fori_loop(0, CHUNK, fill, 0)

    def zacc(i, _):
        k = sid + i * NS

        @pl.when(k < ACC_CH)
        def _():
            pltpu.sync_copy(rows, acc_sh.at[pl.ds(k * CHUNK, CHUNK)])

        return 0

    lax.fori_loop(0, (ACC_CH + NS - 1) // NS, zacc, 0)
    plsc.subcore_barrier()

    def step(j, _):
        base = sid * EPT + j * CHUNK
        pltpu.sync_copy(src_hbm.at[pl.ds(base, CHUNK)], srcv)
        pltpu.sync_copy(dst_hbm.at[pl.ds(base, CHUNK)], dst2.at[0])

        @pl.when(cid == 0)
        def _():
            pltpu.async_copy(xa.at[srcv], rows, sem).wait()

        @pl.when(cid == 1)
        def _():
            pltpu.async_copy(xb.at[srcv], rows, sem).wait()

        pltpu.sync_copy(rows, acc_sh.at[dst2.at[0]], add=True)
        return 0

    lax.fori_loop(0, NCHUNK, step, 0)
    plsc.subcore_barrier()

    def wout(i, _):
        k = sid + i * NS

        @pl.when(k < ACC_CH)
        def _():
            pltpu.sync_copy(acc_sh.at[pl.ds(k * CHUNK, CHUNK)],
                            agg_out.at[cid, pl.ds(k * CHUNK, CHUNK)])

        return 0

    lax.fori_loop(0, (ACC_CH + NS - 1) // NS, wout, 0)


_agg_kernel = functools.partial(
    pl.kernel,
    out_type=jax.ShapeDtypeStruct((NC, ACC_ROWS, HF), jnp.float32),
    mesh=plsc.VectorSubcoreMesh(core_axis_name="c", subcore_axis_name="s"),
    compiler_params=pltpu.CompilerParams(use_tc_tiling_on_sc=False),
    scratch_types=[
        pltpu.VMEM((CHUNK,), jnp.int32),
        pltpu.VMEM((1, CHUNK), jnp.int32),
        pltpu.VMEM((CHUNK, HF), jnp.float32),
        pltpu.VMEM_SHARED((ACC_ROWS, HF), jnp.float32),
        pltpu.SemaphoreType.DMA,
    ],
)(_agg_body)


def _cnt_body(dst_hbm, cnt_out, dst2, cdix, ones, z16, cnt_sh, sem):
    cid = lax.axis_index("c")
    sid = lax.axis_index("s")

    zv = jnp.zeros((L,), jnp.float32)
    ov = jnp.ones((L,), jnp.float32)

    def fill(i, _):
        ones[i] = ov
        z16[i] = zv
        return 0

    lax.fori_loop(0, CHUNK, fill, 0)

    def zacc(i, _):
        k = sid + i * NS

        @pl.when(k < CNT_CH)
        def _():
            pltpu.sync_copy(z16, cnt_sh.at[pl.ds(k * CHUNK, CHUNK)])

        return 0

    lax.fori_loop(0, (CNT_CH + NS - 1) // NS, zacc, 0)
    plsc.subcore_barrier()

    cbase = cid * CNT_H

    def step(j, _):
        base = sid * EPT + j * CHUNK
        pltpu.sync_copy(dst_hbm.at[pl.ds(base, CHUNK)], dst2.at[0])

        for k in range(CHUNK // L):
            d = dst2[0, pl.ds(k * L, L)]
            t = d - cbase
            m = (t >= 0) & (t < CNT_H)
            cdix[0, pl.ds(k * L, L)] = jnp.where(m, t, CNT_H)

        pltpu.sync_copy(ones, cnt_sh.at[cdix.at[0]], add=True)
        return 0

    lax.fori_loop(0, NCHUNK, step, 0)
    plsc.subcore_barrier()

    def wout(i, _):
        k = sid + i * NS

        @pl.when(k < CNT_CH)
        def _():
            pltpu.sync_copy(cnt_sh.at[pl.ds(k * CHUNK, CHUNK)],
                            cnt_out.at[cid, pl.ds(k * CHUNK, CHUNK)])

        return 0

    lax.fori_loop(0, (CNT_CH + NS - 1) // NS, wout, 0)


_cnt_kernel = functools.partial(
    pl.kernel,
    out_type=jax.ShapeDtypeStruct((NC, CNT_ROWS, L), jnp.float32),
    mesh=plsc.VectorSubcoreMesh(core_axis_name="c", subcore_axis_name="s"),
    compiler_params=pltpu.CompilerParams(use_tc_tiling_on_sc=False),
    scratch_types=[
        pltpu.VMEM((1, CHUNK), jnp.int32),
        pltpu.VMEM((1, CHUNK), jnp.int32),
        pltpu.VMEM((CHUNK, L), jnp.float32),
        pltpu.VMEM((CHUNK, L), jnp.float32),
        pltpu.VMEM_SHARED((CNT_ROWS, L), jnp.float32),
        pltpu.SemaphoreType.DMA,
    ],
)(_cnt_body)


BR = 1000


def _tc_body(x_ref, agg_ref, cnt_ref, wi_ref, wu_ref, bias_ref, o_ref):
    cnt1 = cnt_ref[0][:, 0:1]
    mask = cnt1 > 0.0
    rcp = 1.0 / jnp.maximum(cnt1, 1.0)
    agg = jnp.concatenate([agg_ref[0], agg_ref[1]], axis=1)
    mean = agg * rcp
    hi = jnp.dot(x_ref[...], wi_ref[...], preferred_element_type=jnp.float32)
    hu = jnp.dot(mean, wu_ref[...], preferred_element_type=jnp.float32)
    o_ref[...] = hi + bias_ref[0:1, :] + jnp.where(mask, hu + bias_ref[1:2, :], 0.0)


_N_BLK = N_ITEMS // BR
_PLANE_BLKS = CNT_H // BR

_tc_kernel = pl.pallas_call(
    _tc_body,
    out_shape=jax.ShapeDtypeStruct((N_ITEMS, F), jnp.float32),
    grid=(_N_BLK,),
    in_specs=[
        pl.BlockSpec((BR, F), lambda b: (b, 0)),
        pl.BlockSpec((NC, BR, HF), lambda b: (0, b, 0)),
        pl.BlockSpec((1, BR, L), lambda b: (b // _PLANE_BLKS,
                                            b - _PLANE_BLKS * (b // _PLANE_BLKS), 0)),
        pl.BlockSpec((F, F), lambda b: (0, 0)),
        pl.BlockSpec((F, F), lambda b: (0, 0)),
        pl.BlockSpec((2, F), lambda b: (0, 0)),
    ],
    out_specs=pl.BlockSpec((BR, F), lambda b: (b, 0)),
)


def kernel(x_users, x_items, edge_index, W_users, b_users, W_items, b_items):
    xu = x_users.reshape(N_USERS, F)
    xa = xu[:, :HF]
    xb = xu[:, HF:]
    src = edge_index[0].astype(jnp.int32)
    dst = edge_index[1].astype(jnp.int32)
    pad = E_PAD - E
    src_p = jnp.concatenate([src, jnp.zeros((pad,), jnp.int32)])
    dst_p = jnp.concatenate([dst, jnp.full((pad,), N_ITEMS, jnp.int32)])

    agg = _agg_kernel(xa, xb, src_p, dst_p)
    cnt = _cnt_kernel(dst_p)

    pe = jnp.asarray(_positional_encoding_flat())
    eye = jnp.eye(C, dtype=jnp.float32)
    wi_bd = jnp.kron(eye, W_items)
    wu_bd = jnp.kron(eye, W_users)
    bias = jnp.stack([jnp.tile(b_items, C) + pe, jnp.tile(b_users, C) + pe])

    xi = x_items.reshape(N_ITEMS, F)
    out = _tc_kernel(xi, agg, cnt, wi_bd, wu_bd, bias)
    return out.reshape(N_ITEMS, C, D)


# speedup vs baseline: 124.8327x; 1.9328x over previous
import functools

import jax
import jax.numpy as jnp
import numpy as np
from jax import lax
from jax.experimental import pallas as pl
from jax.experimental.pallas import tpu as pltpu
from jax.experimental.pallas import tpu_sc as plsc

N_USERS = 50000
N_ITEMS = 50000
E = 800000
C = 4
FIN = 16
D = 16
F = C * D
HF = F // 2

NC = 2
NS = 16
L = 16

CHUNK = 128
NB = 5
E_PAD = 819200
EPT = E_PAD // NS
NCHUNK = EPT // CHUNK

ACC_ROWS = 50048
ACC_CH = ACC_ROWS // CHUNK
CNL = 8
CCH = E_PAD // (NC * NS * CHUNK)


def _positional_encoding_flat():
    pos = np.arange(C, dtype=np.float32)[:, None]
    i = np.arange(D, dtype=np.float32)[None, :]
    angle = pos / np.power(10000.0, (2.0 * np.floor(i / 2.0)) / D)
    pe = np.zeros((C, D), dtype=np.float32)
    pe[:, 0::2] = np.sin(angle[:, 0::2])
    pe[:, 1::2] = np.cos(angle[:, 1::2])
    return pe.reshape(F)


def _agg_body(xa, xb, src_hbm, dst_hbm, agg_out,
              srcv, dstv, rows, acc_sh, *sems):
    cid = lax.axis_index("c")
    sid = lax.axis_index("s")
    si = sems[:NB]
    sg = sems[NB:2 * NB]
    ss = sems[2 * NB:]

    zv = jnp.zeros((L,), jnp.float32)

    def fill(i, _):
        rows[0, i, pl.ds(0, L)] = zv
        rows[0, i, pl.ds(L, L)] = zv
        return 0

    lax.fori_loop(0, CHUNK, fill, 0)

    def zacc(i, _):
        k = sid + i * NS

        @pl.when(k < ACC_CH)
        def _():
            pltpu.sync_copy(rows.at[0], acc_sh.at[pl.ds(k * CHUNK, CHUNK)])

        return 0

    lax.fori_loop(0, (ACC_CH + NS - 1) // NS, zacc, 0)
    plsc.subcore_barrier()

    ept = sid * EPT

    def fire_idx(j, b):
        pltpu.async_copy(src_hbm.at[pl.ds(ept + j * CHUNK, CHUNK)],
                         srcv.at[b], si[b])
        pltpu.async_copy(dst_hbm.at[pl.ds(ept + j * CHUNK, CHUNK)],
                         dstv.at[b], si[b])

    def wait_idx(j, b):
        pltpu.make_async_copy(src_hbm.at[pl.ds(ept + j * CHUNK, CHUNK)],
                              srcv.at[b], si[b]).wait()
        pltpu.make_async_copy(dst_hbm.at[pl.ds(ept + j * CHUNK, CHUNK)],
                              dstv.at[b], si[b]).wait()

    def fire_gather(j, b):
        @pl.when(cid == 0)
        def _():
            pltpu.async_copy(xa.at[srcv.at[b]], rows.at[b], sg[b])

        @pl.when(cid == 1)
        def _():
            pltpu.async_copy(xb.at[srcv.at[b]], rows.at[b], sg[b])

    def wait_gather(j, b):
        pltpu.make_async_copy(xa.at[srcv.at[b]], rows.at[b], sg[b]).wait()

    def fire_scatter(j, b):
        pltpu.async_copy(rows.at[b], acc_sh.at[dstv.at[b]], ss[b], add=True)

    def wait_scatter(j, b):
        pltpu.make_async_copy(rows.at[b], acc_sh.at[dstv.at[b]],
                              ss[b]).wait()

    for b in range(NB - 1):
        fire_idx(b, b)
    for b in range(NB - 2):
        wait_idx(b, b)
        fire_gather(b, b)

    def group(g, _):
        for b in range(NB):
            j = g * NB + b
            wait_gather(j, b)
            fire_scatter(j, b)
            bp = (b + NB - 1) % NB

            @pl.when(j >= 1)
            def _():
                wait_scatter(j - 1, bp)

            @pl.when(j + NB - 1 < NCHUNK)
            def _():
                fire_idx(j + NB - 1, bp)

            b2 = (b + NB - 2) % NB

            @pl.when(j + NB - 2 < NCHUNK)
            def _():
                wait_idx(j + NB - 2, b2)
                fire_gather(j + NB - 2, b2)

        return 0

    lax.fori_loop(0, NCHUNK // NB, group, 0)
    wait_scatter(NCHUNK - 1, (NCHUNK - 1) % NB)
    plsc.subcore_barrier()

    def wout(i, _):
        k = sid + i * NS

        @pl.when(k < ACC_CH)
        def _():
            pltpu.sync_copy(acc_sh.at[pl.ds(k * CHUNK, CHUNK)],
                            agg_out.at[cid, pl.ds(k * CHUNK, CHUNK)])

        return 0

    lax.fori_loop(0, (ACC_CH + NS - 1) // NS, wout, 0)


_agg_kernel = functools.partial(
    pl.kernel,
    out_type=jax.ShapeDtypeStruct((NC, ACC_ROWS, HF), jnp.float32),
    mesh=plsc.VectorSubcoreMesh(core_axis_name="c", subcore_axis_name="s"),
    compiler_params=pltpu.CompilerParams(use_tc_tiling_on_sc=False),
    scratch_types=[
        pltpu.VMEM((NB, CHUNK), jnp.int32),
        pltpu.VMEM((NB, CHUNK), jnp.int32),
        pltpu.VMEM((NB, CHUNK, HF), jnp.float32),
        pltpu.VMEM_SHARED((ACC_ROWS, HF), jnp.float32),
    ] + [pltpu.SemaphoreType.DMA] * (3 * NB),
)(_agg_body)


def _cnt_body(dst_hbm, oz_hbm, cnt_out, dst_all, ones, z8, cnt_sh, sem):
    cid = lax.axis_index("c")
    sid = lax.axis_index("s")

    pltpu.sync_copy(oz_hbm.at[0], ones)
    pltpu.sync_copy(oz_hbm.at[1], z8)

    def zacc(i, _):
        k = sid + i * NS

        @pl.when(k < ACC_CH)
        def _():
            pltpu.sync_copy(z8, cnt_sh.at[pl.ds(k * CHUNK, CHUNK)])

        return 0

    lax.fori_loop(0, (ACC_CH + NS - 1) // NS, zacc, 0)

    pltpu.sync_copy(dst_hbm.at[cid, sid], dst_all)
    plsc.subcore_barrier()

    DEPTH = 12

    def cfire(j, _):
        pltpu.async_copy(ones, cnt_sh.at[dst_all.at[j]], sem, add=True)

        @pl.when(j >= DEPTH)
        def _():
            jd = jnp.maximum(j - DEPTH, 0)
            pltpu.make_async_copy(ones, cnt_sh.at[dst_all.at[jd]],
                                  sem).wait()

        return 0

    lax.fori_loop(0, CCH, cfire, 0)

    def cdrain(j, _):
        pltpu.make_async_copy(ones, cnt_sh.at[dst_all.at[CCH - DEPTH + j]],
                              sem).wait()
        return 0

    lax.fori_loop(0, DEPTH, cdrain, 0)
    plsc.subcore_barrier()

    def wout(i, _):
        k = sid + i * NS

        @pl.when(k < ACC_CH)
        def _():
            pltpu.sync_copy(cnt_sh.at[pl.ds(k * CHUNK, CHUNK)],
                            cnt_out.at[cid, pl.ds(k * CHUNK, CHUNK)])

        return 0

    lax.fori_loop(0, (ACC_CH + NS - 1) // NS, wout, 0)


_cnt_kernel = functools.partial(
    pl.kernel,
    out_type=jax.ShapeDtypeStruct((NC, ACC_ROWS, CNL), jnp.float32),
    mesh=plsc.VectorSubcoreMesh(core_axis_name="c", subcore_axis_name="s"),
    compiler_params=pltpu.CompilerParams(use_tc_tiling_on_sc=False),
    scratch_types=[
        pltpu.VMEM((CCH, CHUNK), jnp.int32),
        pltpu.VMEM((CHUNK, CNL), jnp.float32),
        pltpu.VMEM((CHUNK, CNL), jnp.float32),
        pltpu.VMEM_SHARED((ACC_ROWS, CNL), jnp.float32),
        pltpu.SemaphoreType.DMA,
    ],
)(_cnt_body)


BR = 1000


def _tc_body(x_ref, agg_ref, cnt_ref, wi_ref, wu_ref, bias_ref, o_ref):
    cnt1 = cnt_ref[0][:, 0:1] + cnt_ref[1][:, 0:1]
    mask = cnt1 > 0.0
    rcp = 1.0 / jnp.maximum(cnt1, 1.0)
    agg = jnp.concatenate([agg_ref[0], agg_ref[1]], axis=1)
    mean = agg * rcp
    hi = jnp.dot(x_ref[...], wi_ref[...], preferred_element_type=jnp.float32)
    hu = jnp.dot(mean, wu_ref[...], preferred_element_type=jnp.float32)
    o_ref[...] = hi + bias_ref[0:1, :] + jnp.where(mask, hu + bias_ref[1:2, :], 0.0)


_N_BLK = N_ITEMS // BR

_tc_kernel = pl.pallas_call(
    _tc_body,
    out_shape=jax.ShapeDtypeStruct((N_ITEMS, F), jnp.float32),
    grid=(_N_BLK,),
    in_specs=[
        pl.BlockSpec((BR, F), lambda b: (b, 0)),
        pl.BlockSpec((NC, BR, HF), lambda b: (0, b, 0)),
        pl.BlockSpec((NC, BR, CNL), lambda b: (0, b, 0)),
        pl.BlockSpec((F, F), lambda b: (0, 0)),
        pl.BlockSpec((F, F), lambda b: (0, 0)),
        pl.BlockSpec((2, F), lambda b: (0, 0)),
    ],
    out_specs=pl.BlockSpec((BR, F), lambda b: (b, 0)),
)


def kernel(x_users, x_items, edge_index, W_users, b_users, W_items, b_items):
    src = edge_index[0].astype(jnp.int32)
    dst = edge_index[1].astype(jnp.int32)
    pad = E_PAD - E
    src_p = jnp.concatenate([src, jnp.zeros((pad,), jnp.int32)])
    dst_p = jnp.concatenate([dst, jnp.full((pad,), N_ITEMS, jnp.int32)])

    oz = jnp.stack([jnp.ones((CHUNK, CNL), jnp.float32),
                    jnp.zeros((CHUNK, CNL), jnp.float32)])
    cnt = _cnt_kernel(dst_p.reshape(NC, NS, CCH, CHUNK), oz)

    xu = x_users.reshape(N_USERS, F)
    xa = xu[:, :HF]
    xb = xu[:, HF:]
    agg = _agg_kernel(xa, xb, src_p, dst_p)

    pe = jnp.asarray(_positional_encoding_flat())
    eye = jnp.eye(C, dtype=jnp.float32)
    wi_bd = jnp.kron(eye, W_items)
    wu_bd = jnp.kron(eye, W_users)
    bias = jnp.stack([jnp.tile(b_items, C) + pe, jnp.tile(b_users, C) + pe])

    xi = x_items.reshape(N_ITEMS, F)
    out = _tc_kernel(xi, agg, cnt, wi_bd, wu_bd, bias)
    return out.reshape(N_ITEMS, C, D)


# speedup vs baseline: 135.4592x; 1.0851x over previous
import functools

import jax
import jax.numpy as jnp
import numpy as np
from jax import lax
from jax.experimental import pallas as pl
from jax.experimental.pallas import tpu as pltpu
from jax.experimental.pallas import tpu_sc as plsc

N_USERS = 50000
N_ITEMS = 50000
E = 800000
C = 4
FIN = 16
D = 16
F = C * D
HF = F // 2

NC = 2
NS = 16
L = 16

CHUNK = 128
NB = 4
E_PAD = 819200
EPT = E_PAD // NS
NCHUNK = EPT // CHUNK

ACC_ROWS = 50048
ACC_CH = ACC_ROWS // CHUNK
CNL = 8
CCH = E_PAD // (NC * NS * CHUNK)


def _positional_encoding_flat():
    pos = np.arange(C, dtype=np.float32)[:, None]
    i = np.arange(D, dtype=np.float32)[None, :]
    angle = pos / np.power(10000.0, (2.0 * np.floor(i / 2.0)) / D)
    pe = np.zeros((C, D), dtype=np.float32)
    pe[:, 0::2] = np.sin(angle[:, 0::2])
    pe[:, 1::2] = np.cos(angle[:, 1::2])
    return pe.reshape(F)


def _agg_body(xa, xb, src_hbm, dst_hbm, agg_out,
              srcv, dstv, rows, acc_sh, *sems):
    cid = lax.axis_index("c")
    sid = lax.axis_index("s")
    si = sems[:NB]
    sg = sems[NB:2 * NB]
    ss = sems[2 * NB:]

    zv = jnp.zeros((L,), jnp.float32)

    def fill(i, _):
        rows[0, i, pl.ds(0, L)] = zv
        rows[0, i, pl.ds(L, L)] = zv
        return 0

    lax.fori_loop(0, CHUNK, fill, 0)

    def zacc(i, _):
        k = sid + i * NS

        @pl.when(k < ACC_CH)
        def _():
            pltpu.sync_copy(rows.at[0], acc_sh.at[pl.ds(k * CHUNK, CHUNK)])

        return 0

    lax.fori_loop(0, (ACC_CH + NS - 1) // NS, zacc, 0)
    plsc.subcore_barrier()

    ept = sid * EPT

    def fire_idx(j, b):
        pltpu.async_copy(src_hbm.at[pl.ds(ept + j * CHUNK, CHUNK)],
                         srcv.at[b], si[b])
        pltpu.async_copy(dst_hbm.at[pl.ds(ept + j * CHUNK, CHUNK)],
                         dstv.at[b], si[b])

    def wait_idx(j, b):
        pltpu.make_async_copy(src_hbm.at[pl.ds(ept + j * CHUNK, CHUNK)],
                              srcv.at[b], si[b]).wait()
        pltpu.make_async_copy(dst_hbm.at[pl.ds(ept + j * CHUNK, CHUNK)],
                              dstv.at[b], si[b]).wait()

    def fire_gather(j, b):
        @pl.when(cid == 0)
        def _():
            pltpu.async_copy(xa.at[srcv.at[b]], rows.at[b], sg[b])

        @pl.when(cid == 1)
        def _():
            pltpu.async_copy(xb.at[srcv.at[b]], rows.at[b], sg[b])

    def wait_gather(j, b):
        pltpu.make_async_copy(xa.at[srcv.at[b]], rows.at[b], sg[b]).wait()

    def fire_scatter(j, b):
        pltpu.async_copy(rows.at[b], acc_sh.at[dstv.at[b]], ss[b], add=True)

    def wait_scatter(j, b):
        pltpu.make_async_copy(rows.at[b], acc_sh.at[dstv.at[b]],
                              ss[b]).wait()

    for b in range(NB - 1):
        fire_idx(b, b)
    for b in range(NB - 2):
        wait_idx(b, b)
        fire_gather(b, b)

    def group(g, _):
        for b in range(NB):
            j = g * NB + b
            wait_gather(j, b)
            fire_scatter(j, b)
            bp = (b + NB - 1) % NB

            @pl.when(j >= 1)
            def _():
                wait_scatter(j - 1, bp)

            @pl.when(j + NB - 1 < NCHUNK)
            def _():
                fire_idx(j + NB - 1, bp)

            b2 = (b + NB - 2) % NB

            @pl.when(j + NB - 2 < NCHUNK)
            def _():
                wait_idx(j + NB - 2, b2)
                fire_gather(j + NB - 2, b2)

        return 0

    lax.fori_loop(0, NCHUNK // NB, group, 0)
    wait_scatter(NCHUNK - 1, (NCHUNK - 1) % NB)
    plsc.subcore_barrier()

    def wout(i, _):
        k = sid + i * NS

        @pl.when(k < ACC_CH)
        def _():
            pltpu.sync_copy(acc_sh.at[pl.ds(k * CHUNK, CHUNK)],
                            agg_out.at[cid, pl.ds(k * CHUNK, CHUNK)])

        return 0

    lax.fori_loop(0, (ACC_CH + NS - 1) // NS, wout, 0)


_agg_kernel = functools.partial(
    pl.kernel,
    out_type=jax.ShapeDtypeStruct((NC, ACC_ROWS, HF), jnp.float32),
    mesh=plsc.VectorSubcoreMesh(core_axis_name="c", subcore_axis_name="s"),
    compiler_params=pltpu.CompilerParams(use_tc_tiling_on_sc=False),
    scratch_types=[
        pltpu.VMEM((NB, CHUNK), jnp.int32),
        pltpu.VMEM((NB, CHUNK), jnp.int32),
        pltpu.VMEM((NB, CHUNK, HF), jnp.float32),
        pltpu.VMEM_SHARED((ACC_ROWS, HF), jnp.float32),
    ] + [pltpu.SemaphoreType.DMA] * (3 * NB),
)(_agg_body)


def _cnt_body(dst_hbm, oz_hbm, cnt_out, dst_all, ones, z8, cnt_sh, sem):
    cid = lax.axis_index("c")
    sid = lax.axis_index("s")

    pltpu.sync_copy(oz_hbm.at[0], ones)
    pltpu.sync_copy(oz_hbm.at[1], z8)

    def zacc(i, _):
        k = sid + i * NS

        @pl.when(k < ACC_CH)
        def _():
            pltpu.sync_copy(z8, cnt_sh.at[pl.ds(k * CHUNK, CHUNK)])

        return 0

    lax.fori_loop(0, (ACC_CH + NS - 1) // NS, zacc, 0)

    pltpu.sync_copy(dst_hbm.at[cid, sid], dst_all)
    plsc.subcore_barrier()

    DEPTH = 12

    def cfire(j, _):
        pltpu.async_copy(ones, cnt_sh.at[dst_all.at[j]], sem, add=True)

        @pl.when(j >= DEPTH)
        def _():
            jd = jnp.maximum(j - DEPTH, 0)
            pltpu.make_async_copy(ones, cnt_sh.at[dst_all.at[jd]],
                                  sem).wait()

        return 0

    lax.fori_loop(0, CCH, cfire, 0)

    def cdrain(j, _):
        pltpu.make_async_copy(ones, cnt_sh.at[dst_all.at[CCH - DEPTH + j]],
                              sem).wait()
        return 0

    lax.fori_loop(0, DEPTH, cdrain, 0)
    plsc.subcore_barrier()

    def wout(i, _):
        k = sid + i * NS

        @pl.when(k < ACC_CH)
        def _():
            pltpu.sync_copy(cnt_sh.at[pl.ds(k * CHUNK, CHUNK)],
                            cnt_out.at[cid, pl.ds(k * CHUNK, CHUNK)])

        return 0

    lax.fori_loop(0, (ACC_CH + NS - 1) // NS, wout, 0)


_cnt_kernel = functools.partial(
    pl.kernel,
    out_type=jax.ShapeDtypeStruct((NC, ACC_ROWS, CNL), jnp.float32),
    mesh=plsc.VectorSubcoreMesh(core_axis_name="c", subcore_axis_name="s"),
    compiler_params=pltpu.CompilerParams(use_tc_tiling_on_sc=False),
    scratch_types=[
        pltpu.VMEM((CCH, CHUNK), jnp.int32),
        pltpu.VMEM((CHUNK, CNL), jnp.float32),
        pltpu.VMEM((CHUNK, CNL), jnp.float32),
        pltpu.VMEM_SHARED((ACC_ROWS, CNL), jnp.float32),
        pltpu.SemaphoreType.DMA,
    ],
)(_cnt_body)


QROWS = ACC_ROWS // 4
QREAL = N_ITEMS // 4
BRQ = 3128
PF = 4 * F


def _sel32():
    s = np.zeros((4, CNL, 4, F), np.float32)
    for u in range(4):
        s[u, 0, u, :] = 1.0
    return s.reshape(4 * CNL, PF)


def _tc_body(xp_ref, aggp_ref, cntp_ref, w2i_ref, w2u_ref, s32_ref,
             biasp_ref, o_ref):
    cnt4 = cntp_ref[0] + cntp_ref[1]
    rcpp = jnp.dot(1.0 / jnp.maximum(cnt4, 1.0), s32_ref[...],
                   preferred_element_type=jnp.float32)
    mskp = jnp.dot((cnt4 > 0.0).astype(jnp.float32), s32_ref[...],
                   preferred_element_type=jnp.float32)
    a = jnp.concatenate([aggp_ref[0], aggp_ref[1]], axis=1)
    hu = jnp.dot(a, w2u_ref[...], preferred_element_type=jnp.float32) * rcpp
    hi = jnp.dot(xp_ref[...], w2i_ref[...], preferred_element_type=jnp.float32)
    o_ref[...] = hi + biasp_ref[0:1, :] + mskp * (hu + biasp_ref[1:2, :])


_tc_kernel = pl.pallas_call(
    _tc_body,
    out_shape=jax.ShapeDtypeStruct((QROWS, PF), jnp.float32),
    grid=(QROWS // BRQ,),
    in_specs=[
        pl.BlockSpec((BRQ, PF), lambda b: (b, 0)),
        pl.BlockSpec((NC, BRQ, 4 * HF), lambda b: (0, b, 0)),
        pl.BlockSpec((NC, BRQ, 4 * CNL), lambda b: (0, b, 0)),
        pl.BlockSpec((PF, PF), lambda b: (0, 0)),
        pl.BlockSpec((PF, PF), lambda b: (0, 0)),
        pl.BlockSpec((4 * CNL, PF), lambda b: (0, 0)),
        pl.BlockSpec((2, PF), lambda b: (0, 0)),
    ],
    out_specs=pl.BlockSpec((BRQ, PF), lambda b: (b, 0)),
)


def kernel(x_users, x_items, edge_index, W_users, b_users, W_items, b_items):
    src = edge_index[0].astype(jnp.int32)
    dst = edge_index[1].astype(jnp.int32)
    pad = E_PAD - E
    src_p = jnp.concatenate([src, jnp.zeros((pad,), jnp.int32)])
    dst_p = jnp.concatenate([dst, jnp.full((pad,), N_ITEMS, jnp.int32)])

    oz = jnp.stack([jnp.ones((CHUNK, CNL), jnp.float32),
                    jnp.zeros((CHUNK, CNL), jnp.float32)])
    cnt = _cnt_kernel(dst_p.reshape(NC, NS, CCH, CHUNK), oz)

    xu = x_users.reshape(N_USERS, F)
    xa = xu[:, :HF]
    xb = xu[:, HF:]
    agg = _agg_kernel(xa, xb, src_p, dst_p)

    pe = jnp.asarray(_positional_encoding_flat())
    eye = jnp.eye(C, dtype=jnp.float32)
    wi_bd = jnp.kron(eye, W_items)
    wu_bd = jnp.kron(eye, W_users)
    w2i = jnp.einsum('uv,fo->ufvo', eye, wi_bd).reshape(PF, PF)
    w2u = jnp.einsum('uv,hfo->hufvo', eye,
                     wu_bd.reshape(NC, HF, F)).reshape(PF, PF)
    biasp = jnp.stack([jnp.tile(jnp.tile(b_items, C) + pe, 4),
                       jnp.tile(jnp.tile(b_users, C) + pe, 4)])
    s32 = jnp.asarray(_sel32())

    xi_p = jnp.pad(x_items.reshape(QREAL, PF), ((0, QROWS - QREAL), (0, 0)))
    aggp = agg.reshape(NC, QROWS, 4 * HF)
    cntp = cnt.reshape(NC, QROWS, 4 * CNL)
    outp = _tc_kernel(xi_p, aggp, cntp, w2i, w2u, s32, biasp)
    return outp[:QREAL].reshape(N_ITEMS, C, D)
